# baseline (speedup 1.0000x reference)
import functools

import jax
import jax.numpy as jnp
from jax import lax
from jax.experimental import pallas as pl
from jax.experimental.pallas import tpu as pltpu
from jax.experimental.pallas import tpu_sc as plsc

HIDDEN = 4096
NUM_EXPERTS = 64
TOP_K = 8
BLOCK_T = 512

_NC = 2
_NS = 16
_NW = _NC * _NS
_LANES = 16

_SC_CHUNK = 128
_N_SLICES = 8


def _matmul_body(h_ref, w_ref, logits_ref, logits_t_ref):
    logits = lax.dot_general(h_ref[...], w_ref[...],
                             (((1,), (1,)), ((), ())),
                             preferred_element_type=jnp.float32)
    logits_ref[...] = logits
    logits_t_ref[...] = jnp.swapaxes(logits, 0, 1)


def _tc_logits(h_flat, w, s, n_slices):
    n_tokens = h_flat.shape[0]
    sl_tokens = n_tokens // n_slices
    blk0 = s * (sl_tokens // BLOCK_T)
    return pl.pallas_call(
        _matmul_body,
        grid=(sl_tokens // BLOCK_T,),
        in_specs=[
            pl.BlockSpec((BLOCK_T, HIDDEN), lambda i: (blk0 + i, 0)),
            pl.BlockSpec((NUM_EXPERTS, HIDDEN), lambda i: (0, 0)),
        ],
        out_specs=[
            pl.BlockSpec((BLOCK_T, NUM_EXPERTS), lambda i: (i, 0)),
            pl.BlockSpec((NUM_EXPERTS, BLOCK_T), lambda i: (0, i)),
        ],
        out_shape=[
            jax.ShapeDtypeStruct((sl_tokens, NUM_EXPERTS), jnp.float32),
            jax.ShapeDtypeStruct((NUM_EXPERTS, sl_tokens), jnp.float32),
        ],
        compiler_params=pltpu.CompilerParams(
            dimension_semantics=("arbitrary",),
        ),
    )(h_flat, w)


def _sc_topk_body(logits_t_hbm, vals_hbm, idx_hbm, tr, outv, outi):
    sl_tokens = logits_t_hbm.shape[1]
    per_worker = sl_tokens // _NW
    n_chunks = per_worker // _SC_CHUNK
    wid = lax.axis_index("c") * _NS + lax.axis_index("s")
    lane = lax.iota(jnp.int32, _LANES)
    neg_inf = jnp.full((_LANES,), -jnp.inf, jnp.float32)

    def argmax_tree(pairs):
        while len(pairs) > 1:
            nxt = []
            for i in range(0, len(pairs) - 1, 2):
                (va, ia), (vb, ib) = pairs[i], pairs[i + 1]
                gt = vb > va
                nxt.append((jnp.where(gt, vb, va), jnp.where(gt, ib, ia)))
            if len(pairs) % 2:
                nxt.append(pairs[-1])
            pairs = nxt
        return pairs[0]

    def process_group(g, _):
        goff = g * _LANES
        rows = goff + lane
        best_vals = []
        best_idxs = []
        for _k in range(TOP_K):
            groups = []
            for c in range(0, NUM_EXPERTS, 8):
                leaves = []
                for e in range(c, c + 8):
                    v = tr[e, pl.ds(goff, _LANES)]
                    leaves.append((v, jnp.full((_LANES,), e, jnp.int32)))
                groups.append(argmax_tree(leaves))
            bv, bi = argmax_tree(groups)
            plsc.store_scatter(tr, [bi, rows], neg_inf)
            best_vals.append(bv)
            best_idxs.append(bi)
        exps = [jnp.exp(v - best_vals[0]) for v in best_vals]
        denom = exps[0]
        for e_ in exps[1:]:
            denom = denom + e_
        for k in range(TOP_K):
            colk = jnp.full((_LANES,), k, jnp.int32)
            plsc.store_scatter(outv, [rows, colk], exps[k] / denom)
            plsc.store_scatter(outi, [rows, colk], best_idxs[k])
        return _

    def process_chunk(ci, _):
        base = wid * per_worker + ci * _SC_CHUNK
        pltpu.sync_copy(logits_t_hbm.at[:, pl.ds(base, _SC_CHUNK)], tr)
        lax.fori_loop(0, _SC_CHUNK // _LANES, process_group, None)
        pltpu.sync_copy(outv, vals_hbm.at[pl.ds(base, _SC_CHUNK)])
        pltpu.sync_copy(outi, idx_hbm.at[pl.ds(base, _SC_CHUNK)])
        return _

    lax.fori_loop(0, n_chunks, process_chunk, None)


def _sc_topk(logits_t):
    sl_tokens = logits_t.shape[1]
    mesh = plsc.VectorSubcoreMesh(core_axis_name="c", subcore_axis_name="s")
    return pl.kernel(
        _sc_topk_body,
        out_type=[
            jax.ShapeDtypeStruct((sl_tokens, TOP_K), jnp.float32),
            jax.ShapeDtypeStruct((sl_tokens, TOP_K), jnp.int32),
        ],
        mesh=mesh,
        compiler_params=pltpu.CompilerParams(needs_layout_passes=False),
        scratch_types=[
            pltpu.VMEM((NUM_EXPERTS, _SC_CHUNK), jnp.float32),
            pltpu.VMEM((_SC_CHUNK, TOP_K), jnp.float32),
            pltpu.VMEM((_SC_CHUNK, TOP_K), jnp.int32),
        ],
    )(logits_t)


@jax.jit
def kernel(hidden_states, weight):
    h_flat = hidden_states.reshape(-1, hidden_states.shape[-1])
    n_tokens = h_flat.shape[0]
    logits_parts, vals_parts, idx_parts = [], [], []
    for s in range(_N_SLICES):
        logits_s, logits_t_s = _tc_logits(h_flat, weight, s, _N_SLICES)
        vals_s, idx_s = _sc_topk(logits_t_s)
        logits_parts.append(logits_s)
        vals_parts.append(vals_s)
        idx_parts.append(idx_s)
    logits = jnp.concatenate(logits_parts, axis=0)
    vals = jnp.concatenate(vals_parts, axis=0)
    idx = jnp.concatenate(idx_parts, axis=0)
    return (logits, vals.astype(hidden_states.dtype), idx)


# speedup vs baseline: 1.0930x; 1.0930x over previous
import functools

import jax
import jax.numpy as jnp
from jax import lax
from jax.experimental import pallas as pl
from jax.experimental.pallas import tpu as pltpu
from jax.experimental.pallas import tpu_sc as plsc

HIDDEN = 4096
NUM_EXPERTS = 64
TOP_K = 8
BLOCK_T = 512

_NC = 2
_NS = 16
_NW = _NC * _NS
_LANES = 16

_SC_CHUNK = 256
_N_SLICES = 4


def _matmul_body(h_ref, w_ref, logits_ref, logits_t_ref):
    logits = lax.dot_general(h_ref[...], w_ref[...],
                             (((1,), (1,)), ((), ())),
                             preferred_element_type=jnp.float32)
    logits_ref[...] = logits
    logits_t_ref[...] = jnp.swapaxes(logits, 0, 1)


def _tc_logits(h_flat, w, s, n_slices):
    n_tokens = h_flat.shape[0]
    sl_tokens = n_tokens // n_slices
    blk0 = s * (sl_tokens // BLOCK_T)
    return pl.pallas_call(
        _matmul_body,
        grid=(sl_tokens // BLOCK_T,),
        in_specs=[
            pl.BlockSpec((BLOCK_T, HIDDEN), lambda i: (blk0 + i, 0)),
            pl.BlockSpec((NUM_EXPERTS, HIDDEN), lambda i: (0, 0)),
        ],
        out_specs=[
            pl.BlockSpec((BLOCK_T, NUM_EXPERTS), lambda i: (i, 0)),
            pl.BlockSpec((NUM_EXPERTS, BLOCK_T), lambda i: (0, i)),
        ],
        out_shape=[
            jax.ShapeDtypeStruct((sl_tokens, NUM_EXPERTS), jnp.float32),
            jax.ShapeDtypeStruct((NUM_EXPERTS, sl_tokens), jnp.float32),
        ],
        compiler_params=pltpu.CompilerParams(
            dimension_semantics=("arbitrary",),
        ),
    )(h_flat, w)


def _sc_topk_body(logits_t_hbm, vals_hbm, idx_hbm, tr, outv, outi):
    sl_tokens = logits_t_hbm.shape[1]
    per_worker = sl_tokens // _NW
    n_chunks = per_worker // _SC_CHUNK
    wid = lax.axis_index("c") * _NS + lax.axis_index("s")
    lane = lax.iota(jnp.int32, _LANES)
    neg_inf = jnp.full((_LANES,), -jnp.inf, jnp.float32)

    def argmax_tree(pairs):
        while len(pairs) > 1:
            nxt = []
            for i in range(0, len(pairs) - 1, 2):
                (va, ia), (vb, ib) = pairs[i], pairs[i + 1]
                gt = vb > va
                nxt.append((jnp.where(gt, vb, va), jnp.where(gt, ib, ia)))
            if len(pairs) % 2:
                nxt.append(pairs[-1])
            pairs = nxt
        return pairs[0]

    def process_group(g, _):
        goff = g * _LANES
        rows = goff + lane
        best_vals = []
        best_idxs = []
        for _k in range(TOP_K):
            groups = []
            for c in range(0, NUM_EXPERTS, 8):
                leaves = []
                for e in range(c, c + 8):
                    v = tr[e, pl.ds(goff, _LANES)]
                    leaves.append((v, jnp.full((_LANES,), e, jnp.int32)))
                groups.append(argmax_tree(leaves))
            bv, bi = argmax_tree(groups)
            plsc.store_scatter(tr, [bi, rows], neg_inf)
            best_vals.append(bv)
            best_idxs.append(bi)
        exps = [jnp.exp(v - best_vals[0]) for v in best_vals]
        denom = exps[0]
        for e_ in exps[1:]:
            denom = denom + e_
        for k in range(TOP_K):
            colk = jnp.full((_LANES,), k, jnp.int32)
            plsc.store_scatter(outv, [rows, colk], exps[k] / denom)
            plsc.store_scatter(outi, [rows, colk], best_idxs[k])
        return _

    def process_chunk(ci, _):
        base = wid * per_worker + ci * _SC_CHUNK
        pltpu.sync_copy(logits_t_hbm.at[:, pl.ds(base, _SC_CHUNK)], tr)
        lax.fori_loop(0, _SC_CHUNK // _LANES, process_group, None)
        pltpu.sync_copy(outv, vals_hbm.at[pl.ds(base, _SC_CHUNK)])
        pltpu.sync_copy(outi, idx_hbm.at[pl.ds(base, _SC_CHUNK)])
        return _

    lax.fori_loop(0, n_chunks, process_chunk, None)


def _sc_topk(logits_t):
    sl_tokens = logits_t.shape[1]
    mesh = plsc.VectorSubcoreMesh(core_axis_name="c", subcore_axis_name="s")
    return pl.kernel(
        _sc_topk_body,
        out_type=[
            jax.ShapeDtypeStruct((sl_tokens, TOP_K), jnp.float32),
            jax.ShapeDtypeStruct((sl_tokens, TOP_K), jnp.int32),
        ],
        mesh=mesh,
        compiler_params=pltpu.CompilerParams(needs_layout_passes=False),
        scratch_types=[
            pltpu.VMEM((NUM_EXPERTS, _SC_CHUNK), jnp.float32),
            pltpu.VMEM((_SC_CHUNK, TOP_K), jnp.float32),
            pltpu.VMEM((_SC_CHUNK, TOP_K), jnp.int32),
        ],
    )(logits_t)


@jax.jit
def kernel(hidden_states, weight):
    h_flat = hidden_states.reshape(-1, hidden_states.shape[-1])
    n_tokens = h_flat.shape[0]
    logits_parts, vals_parts, idx_parts = [], [], []
    for s in range(_N_SLICES):
        logits_s, logits_t_s = _tc_logits(h_flat, weight, s, _N_SLICES)
        vals_s, idx_s = _sc_topk(logits_t_s)
        logits_parts.append(logits_s)
        vals_parts.append(vals_s)
        idx_parts.append(idx_s)
    logits = jnp.concatenate(logits_parts, axis=0)
    vals = jnp.concatenate(vals_parts, axis=0)
    idx = jnp.concatenate(idx_parts, axis=0)
    return (logits, vals.astype(hidden_states.dtype), idx)


# speedup vs baseline: 1.1166x; 1.0216x over previous
import functools

import jax
import jax.numpy as jnp
from jax import lax
from jax.experimental import pallas as pl
from jax.experimental.pallas import tpu as pltpu
from jax.experimental.pallas import tpu_sc as plsc

HIDDEN = 4096
NUM_EXPERTS = 64
TOP_K = 8
BLOCK_T = 512

_NC = 2
_NS = 16
_NW = _NC * _NS
_LANES = 16

_SC_CHUNK = 256
_N_SLICES = 4


def _matmul_body(h_ref, w_ref, logits_ref, logits_t_ref):
    logits = lax.dot_general(h_ref[...], w_ref[...],
                             (((1,), (1,)), ((), ())),
                             preferred_element_type=jnp.float32)
    logits_ref[...] = logits
    logits_t_ref[...] = jnp.swapaxes(logits, 0, 1)


def _tc_logits(h_flat, w, s, n_slices):
    n_tokens = h_flat.shape[0]
    sl_tokens = n_tokens // n_slices
    blk0 = s * (sl_tokens // BLOCK_T)
    return pl.pallas_call(
        _matmul_body,
        grid=(sl_tokens // BLOCK_T,),
        in_specs=[
            pl.BlockSpec((BLOCK_T, HIDDEN), lambda i: (blk0 + i, 0)),
            pl.BlockSpec((NUM_EXPERTS, HIDDEN), lambda i: (0, 0)),
        ],
        out_specs=[
            pl.BlockSpec((BLOCK_T, NUM_EXPERTS), lambda i: (i, 0)),
            pl.BlockSpec((NUM_EXPERTS, BLOCK_T), lambda i: (0, i)),
        ],
        out_shape=[
            jax.ShapeDtypeStruct((sl_tokens, NUM_EXPERTS), jnp.float32),
            jax.ShapeDtypeStruct((NUM_EXPERTS, sl_tokens), jnp.float32),
        ],
        compiler_params=pltpu.CompilerParams(
            dimension_semantics=("arbitrary",),
        ),
    )(h_flat, w)


def _sc_topk_body(logits_t_hbm, vals_hbm, idx_hbm, tr, outv, outi):
    sl_tokens = logits_t_hbm.shape[1]
    per_worker = sl_tokens // _NW
    n_chunks = per_worker // _SC_CHUNK
    wid = lax.axis_index("c") * _NS + lax.axis_index("s")
    lane = lax.iota(jnp.int32, _LANES)
    neg_inf = jnp.full((_LANES,), -jnp.inf, jnp.float32)

    def argmax_tree(pairs):
        while len(pairs) > 1:
            nxt = []
            for i in range(0, len(pairs) - 1, 2):
                (va, ia), (vb, ib) = pairs[i], pairs[i + 1]
                gt = vb > va
                nxt.append((jnp.where(gt, vb, va), jnp.where(gt, ib, ia)))
            if len(pairs) % 2:
                nxt.append(pairs[-1])
            pairs = nxt
        return pairs[0]

    def process_group(g, _):
        goff = g * _LANES
        rows = goff + lane
        best_vals = []
        best_idxs = []
        for _k in range(TOP_K):
            groups = []
            for c in range(0, NUM_EXPERTS, 8):
                leaves = []
                for e in range(c, c + 8):
                    v = tr[e, pl.ds(goff, _LANES)]
                    leaves.append((v, jnp.full((_LANES,), e, jnp.int32)))
                groups.append(argmax_tree(leaves))
            bv, bi = argmax_tree(groups)
            plsc.store_scatter(tr, [bi, rows], neg_inf)
            best_vals.append(bv)
            best_idxs.append(bi)
        exps = [jnp.exp(v - best_vals[0]) for v in best_vals]
        denom = exps[0]
        for e_ in exps[1:]:
            denom = denom + e_
        for k in range(TOP_K):
            colk = jnp.full((_LANES,), k, jnp.int32)
            plsc.store_scatter(outv, [rows, colk], exps[k] / denom)
            plsc.store_scatter(outi, [rows, colk], best_idxs[k])
        return _

    def process_chunk(ci, _):
        base = wid * per_worker + ci * _SC_CHUNK
        pltpu.sync_copy(logits_t_hbm.at[:, pl.ds(base, _SC_CHUNK)], tr)
        lax.fori_loop(0, _SC_CHUNK // _LANES, process_group, None)
        pltpu.sync_copy(outv, vals_hbm.at[pl.ds(base, _SC_CHUNK)])
        pltpu.sync_copy(outi, idx_hbm.at[pl.ds(base, _SC_CHUNK)])
        return _

    lax.fori_loop(0, n_chunks, process_chunk, None)


def _sc_topk(logits_t):
    sl_tokens = logits_t.shape[1]
    mesh = plsc.VectorSubcoreMesh(core_axis_name="c", subcore_axis_name="s")
    return pl.kernel(
        _sc_topk_body,
        out_type=[
            jax.ShapeDtypeStruct((sl_tokens, TOP_K), jnp.float32),
            jax.ShapeDtypeStruct((sl_tokens, TOP_K), jnp.int32),
        ],
        mesh=mesh,
        compiler_params=pltpu.CompilerParams(needs_layout_passes=False),
        scratch_types=[
            pltpu.VMEM((NUM_EXPERTS, _SC_CHUNK), jnp.float32),
            pltpu.VMEM((_SC_CHUNK, TOP_K), jnp.float32),
            pltpu.VMEM((_SC_CHUNK, TOP_K), jnp.int32),
        ],
    )(logits_t)


@jax.jit
def kernel(hidden_states, weight):
    h_flat = hidden_states.reshape(-1, hidden_states.shape[-1])
    n_tokens = h_flat.shape[0]
    logits_parts, vals_parts, idx_parts = [], [], []
    for s in range(_N_SLICES):
        logits_s, logits_t_s = _tc_logits(h_flat, weight, s, _N_SLICES)
        vals_s, idx_s = _sc_topk(logits_t_s)
        logits_parts.append(logits_s)
        vals_parts.append(vals_s)
        idx_parts.append(idx_s)
    logits = jnp.stack(logits_parts, axis=0).reshape(n_tokens, NUM_EXPERTS)
    vals = jnp.concatenate(vals_parts, axis=0)
    idx = jnp.concatenate(idx_parts, axis=0)
    return (logits, vals.astype(hidden_states.dtype), idx)
